# baseline (speedup 1.0000x reference)
import functools

import jax
import jax.numpy as jnp
from jax import lax
from jax.experimental import pallas as pl
from jax.experimental.pallas import tpu as pltpu
from jax.experimental.pallas import tpu_sc as plsc

_LANES = 16
_IDXB = 14


def _sc_spmm(N, D, EP):
    CH = 64
    NC, NS = 2, 16
    NW = NC * NS
    NCHW = EP // (NW * CH)
    BUFS = 4
    GROUPS = NCHW // BUFS
    REM = NCHW - GROUPS * BUFS
    assert EP == NW * NCHW * CH and NCHW % 8 == 0 and REM < BUFS
    RPS = (N // NS) // 8 * 8
    TAIL = N - NS * RPS
    assert RPS % 8 == 0 and TAIL % 8 == 0 and TAIL <= CH

    mesh = plsc.VectorSubcoreMesh(core_axis_name="c", subcore_axis_name="s")

    @functools.partial(
        pl.kernel,
        out_type=jax.ShapeDtypeStruct((NC, N, D), jnp.float32),
        mesh=mesh,
        compiler_params=pltpu.CompilerParams(needs_layout_passes=False),
        scratch_types=[
            pltpu.VMEM((NCHW // 2, 2 * CH), jnp.int32),
            pltpu.VMEM((NCHW // 4, 2 * CH), jnp.int32),
            [pltpu.VMEM((CH,), jnp.int32) for _ in range(BUFS)],
            [pltpu.VMEM((CH,), jnp.int32) for _ in range(BUFS)],
            [pltpu.VMEM((CH, D), jnp.float32) for _ in range(BUFS)],
            pltpu.VMEM_SHARED((N, D), jnp.float32),
            [pltpu.SemaphoreType.DMA for _ in range(BUFS)],
            [pltpu.SemaphoreType.DMA for _ in range(BUFS)],
            pltpu.SemaphoreType.DMA,
        ],
    )
    def spmm_kernel(packed_hbm, vals_hbm, x_hbm, out_hbm,
                    packed_v, vals_v, rows_c, cols_c, gx, acc,
                    gsem, ssem, stsem):
        cid = lax.axis_index("c")
        sid = lax.axis_index("s")
        wid = sid * NC + cid
        wbase = wid * (NCHW // 2)

        def meta_sl(ref, c, j):
            return ref[c // 2, pl.ds((c % 2) * CH + j * _LANES, _LANES)]

        st0 = pltpu.async_copy(packed_hbm.at[pl.ds(wbase, NCHW // 2)],
                               packed_v, stsem)
        st1 = pltpu.async_copy(vals_hbm.at[pl.ds(wid * (NCHW // 4),
                                                 NCHW // 4)],
                               vals_v, stsem)

        zero16 = jnp.zeros((_LANES,), jnp.float32)

        def zbody(r, carry):
            for g in range(D // _LANES):
                gx[0][r, pl.ds(g * _LANES, _LANES)] = zero16
            return carry

        lax.fori_loop(0, CH, zbody, 0)
        rbase = sid * RPS
        for j in range(RPS // CH):
            pltpu.sync_copy(gx[0], acc.at[pl.ds(rbase + j * CH, CH)])
        zrem = RPS - (RPS // CH) * CH
        if zrem:
            pltpu.sync_copy(gx[0].at[pl.ds(0, zrem)],
                            acc.at[pl.ds(rbase + RPS - zrem, zrem)])

        @pl.when(sid == NS - 1)
        def _():
            pltpu.sync_copy(gx[0].at[pl.ds(0, TAIL)],
                            acc.at[pl.ds(NS * RPS, TAIL)])

        st0.wait()
        st1.wait()
        plsc.subcore_barrier()

        def unpack(c, b):
            for v in range(CH // _LANES):
                sl = pl.ds(v * _LANES, _LANES)
                pk = meta_sl(packed_v, c, v)
                cols_c[b][sl] = jnp.bitwise_and(pk, (1 << _IDXB) - 1)
                rows_c[b][sl] = lax.shift_right_logical(pk, _IDXB)

        def issue_gather(c, b):
            return pltpu.async_copy(x_hbm.at[cols_c[b]], gx[b], gsem[b])

        def wait_gather(b):
            pltpu.make_async_copy(x_hbm.at[cols_c[b]], gx[b], gsem[b]).wait()

        def issue_scatter(b):
            return pltpu.async_copy(gx[b], acc.at[rows_c[b]], ssem[b],
                                    add=True)

        def wait_scatter(b):
            pltpu.make_async_copy(gx[b], acc.at[rows_c[b]], ssem[b]).wait()

        def scale(c, b):
            hmask = jnp.full((_LANES,), -(1 << 16), jnp.int32)

            def jbody(j, carry):
                q = c * (CH // 2) + (j // 2) * _LANES
                vvec = vals_v[q // (2 * CH), pl.ds(q % (2 * CH), _LANES)]
                lbase = (j % 2) * 8
                for e16 in range(_LANES):
                    idx = (jnp.full((_LANES,), e16 // 2, jnp.int32)
                           + lbase).reshape(_LANES, 1)
                    pair = lax.gather(
                        vvec, idx,
                        lax.GatherDimensionNumbers(
                            offset_dims=(), collapsed_slice_dims=(0,),
                            start_index_map=(0,)),
                        (1,),
                        mode=lax.GatherScatterMode.PROMISE_IN_BOUNDS)
                    if e16 % 2 == 0:
                        vb = plsc.bitcast(lax.shift_left(pair, 16),
                                          jnp.float32)
                    else:
                        vb = plsc.bitcast(jnp.bitwise_and(pair, hmask),
                                          jnp.float32)
                    e = j * _LANES + e16
                    for g in range(D // _LANES):
                        sl = pl.ds(g * _LANES, _LANES)
                        gx[b][e, sl] = gx[b][e, sl] * vb
                return carry

            lax.fori_loop(0, CH // _LANES, jbody, 0)

        def slot(c, b):
            wait_gather(b)
            bn = (b + 3) % BUFS

            @pl.when(c >= 1)
            def _():
                wait_scatter(bn)

            @pl.when(c + 3 < NCHW)
            def _():
                unpack(c + 3, bn)
                issue_gather(c + 3, bn)

            scale(c, b)
            issue_scatter(b)

        for pb in range(BUFS - 1):
            unpack(pb, pb)
            issue_gather(pb, pb)

        def gbody(g, carry):
            c0 = g * BUFS
            for b in range(BUFS):
                slot(c0 + b, b)
            return carry

        lax.fori_loop(0, GROUPS, gbody, 0)
        for r in range(REM):
            slot(GROUPS * BUFS + r, r)

        wait_scatter((NCHW - 1) % BUFS)
        plsc.subcore_barrier()

        pltpu.sync_copy(acc.at[pl.ds(rbase, RPS)],
                        out_hbm.at[cid, pl.ds(rbase, RPS)])

        @pl.when(sid == NS - 1)
        def _():
            pltpu.sync_copy(acc.at[pl.ds(NS * RPS, TAIL)],
                            out_hbm.at[cid, pl.ds(NS * RPS, TAIL)])

    return spmm_kernel


def _dense(h, p, Wa, Wb, b, relu, res):
    N, D = h.shape
    BN = 1000

    def body(h_ref, p_ref, wa_ref, wb_ref, b_ref, o_ref):
        z = p_ref[0] + p_ref[1]
        acc = jnp.dot(h_ref[...], wa_ref[...],
                      preferred_element_type=jnp.float32)
        acc = acc + jnp.dot(z, wb_ref[...],
                            preferred_element_type=jnp.float32)
        acc = acc + b_ref[...]
        if res:
            acc = acc + h_ref[...]
        if relu:
            acc = jnp.maximum(acc, 0.0)
        o_ref[...] = acc

    return pl.pallas_call(
        body,
        grid=(N // BN,),
        in_specs=[
            pl.BlockSpec((BN, D), lambda i: (i, 0)),
            pl.BlockSpec((2, BN, D), lambda i: (0, i, 0)),
            pl.BlockSpec((D, D), lambda i: (0, 0)),
            pl.BlockSpec((D, D), lambda i: (0, 0)),
            pl.BlockSpec((1, D), lambda i: (0, 0)),
        ],
        out_specs=pl.BlockSpec((BN, D), lambda i: (i, 0)),
        out_shape=jax.ShapeDtypeStruct((N, D), jnp.float32),
    )(h, p, Wa, Wb, b.reshape(1, D))


def kernel(x, laplacian_indices, laplacian_values, W0, b0, W1, b1, W2, b2):
    N, D = x.shape
    E = laplacian_values.shape[0]
    assert N <= (1 << _IDXB)
    rows = laplacian_indices[0]
    cols = laplacian_indices[1]

    CH, NW = 64, 32
    NCHW = -(-E // (NW * CH))
    NCHW = -(-NCHW // 8) * 8
    EP = NW * NCHW * CH
    pad = EP - E
    if pad:
        pidx = (jnp.arange(pad, dtype=jnp.int32) * 37) % N
        rows_p = jnp.concatenate([rows, pidx])
        cols_p = jnp.concatenate([cols, pidx])
        vals_p = jnp.concatenate(
            [laplacian_values, jnp.zeros((pad,), jnp.float32)])
    else:
        rows_p, cols_p, vals_p = rows, cols, laplacian_values
    packed = ((rows_p << _IDXB) | cols_p).reshape(EP // (2 * CH), 2 * CH)
    vals2 = lax.bitcast_convert_type(
        vals_p.astype(jnp.bfloat16).reshape(EP // 2, 2),
        jnp.int32).reshape(EP // (4 * CH), 2 * CH)

    Wa0, Wb0 = W0[:, 0::2].T, W0[:, 1::2].T
    Wa1, Wb1 = W1[:, 0::2].T, W1[:, 1::2].T
    Wa2, Wb2 = W2[:, 0::2].T, W2[:, 1::2].T

    spmm = _sc_spmm(N, D, EP)

    p = spmm(packed, vals2, x)
    h = _dense(x, p, Wa0, Wb0, b0, relu=True, res=False)
    p = spmm(packed, vals2, h)
    h = _dense(h, p, Wa1, Wb1, b1, relu=True, res=True)
    p = spmm(packed, vals2, h)
    out = _dense(h, p, Wa2, Wb2, b2, relu=False, res=False)
    return out


# speedup vs baseline: 1.2709x; 1.2709x over previous
import functools

import jax
import jax.numpy as jnp
from jax import lax
from jax.experimental import pallas as pl
from jax.experimental.pallas import tpu as pltpu
from jax.experimental.pallas import tpu_sc as plsc

_LANES = 16
_IDXB = 14


def _sc_spmm(N, D, EP):
    CH = 64
    NC, NS = 2, 16
    NW = NC * NS
    NCHW = EP // (NW * CH)
    BUFS = 3
    GROUPS = NCHW // BUFS
    REM = NCHW - GROUPS * BUFS
    assert EP == NW * NCHW * CH and NCHW % 8 == 0 and REM < BUFS
    RPS = (N // NS) // 8 * 8
    TAIL = N - NS * RPS
    assert RPS % 8 == 0 and TAIL % 8 == 0 and TAIL <= CH

    mesh = plsc.VectorSubcoreMesh(core_axis_name="c", subcore_axis_name="s")

    @functools.partial(
        pl.kernel,
        out_type=jax.ShapeDtypeStruct((NC, N, D), jnp.float32),
        mesh=mesh,
        scratch_types=[
            pltpu.VMEM((NCHW // 2, 2 * CH), jnp.int32),
            pltpu.VMEM((NCHW // 2, 2 * CH), jnp.float32),
            [pltpu.VMEM((CH,), jnp.int32) for _ in range(BUFS)],
            [pltpu.VMEM((CH,), jnp.int32) for _ in range(BUFS)],
            [pltpu.VMEM((CH, D), jnp.float32) for _ in range(BUFS)],
            pltpu.VMEM_SHARED((N, D), jnp.float32),
            [pltpu.SemaphoreType.DMA for _ in range(BUFS)],
            [pltpu.SemaphoreType.DMA for _ in range(BUFS)],
            pltpu.SemaphoreType.DMA,
        ],
    )
    def spmm_kernel(packed_hbm, vals_hbm, x_hbm, out_hbm,
                    packed_v, vals_v, rows_c, cols_c, gx, acc,
                    gsem, ssem, stsem):
        cid = lax.axis_index("c")
        sid = lax.axis_index("s")
        wid = sid * NC + cid
        wbase = wid * (NCHW // 2)

        def meta_sl(ref, c, j):
            return ref[c // 2, pl.ds((c % 2) * CH + j * _LANES, _LANES)]

        st0 = pltpu.async_copy(packed_hbm.at[pl.ds(wbase, NCHW // 2)],
                               packed_v, stsem)
        st1 = pltpu.async_copy(vals_hbm.at[pl.ds(wbase, NCHW // 2)],
                               vals_v, stsem)

        zero16 = jnp.zeros((_LANES,), jnp.float32)

        def zbody(r, carry):
            for g in range(D // _LANES):
                gx[0][r, pl.ds(g * _LANES, _LANES)] = zero16
            return carry

        lax.fori_loop(0, CH, zbody, 0)
        rbase = sid * RPS
        for j in range(RPS // CH):
            pltpu.sync_copy(gx[0], acc.at[pl.ds(rbase + j * CH, CH)])
        zrem = RPS - (RPS // CH) * CH
        if zrem:
            pltpu.sync_copy(gx[0].at[pl.ds(0, zrem)],
                            acc.at[pl.ds(rbase + RPS - zrem, zrem)])

        @pl.when(sid == NS - 1)
        def _():
            pltpu.sync_copy(gx[0].at[pl.ds(0, TAIL)],
                            acc.at[pl.ds(NS * RPS, TAIL)])

        st0.wait()
        st1.wait()
        plsc.subcore_barrier()

        def unpack(c, b):
            for v in range(CH // _LANES):
                sl = pl.ds(v * _LANES, _LANES)
                pk = meta_sl(packed_v, c, v)
                cols_c[b][sl] = jnp.bitwise_and(pk, (1 << _IDXB) - 1)
                rows_c[b][sl] = lax.shift_right_logical(pk, _IDXB)

        def issue_gather(c, b):
            return pltpu.async_copy(x_hbm.at[cols_c[b]], gx[b], gsem[b])

        def wait_gather(b):
            pltpu.make_async_copy(x_hbm.at[cols_c[b]], gx[b], gsem[b]).wait()

        def issue_scatter(b):
            return pltpu.async_copy(gx[b], acc.at[rows_c[b]], ssem[b],
                                    add=True)

        def wait_scatter(b):
            pltpu.make_async_copy(gx[b], acc.at[rows_c[b]], ssem[b]).wait()

        def scale(c, b):
            def jbody(j, carry):
                vvec = meta_sl(vals_v, c, j)
                for e16 in range(_LANES):
                    idx = jnp.full((_LANES, 1), e16, jnp.int32)
                    vb = lax.gather(
                        vvec, idx,
                        lax.GatherDimensionNumbers(
                            offset_dims=(), collapsed_slice_dims=(0,),
                            start_index_map=(0,)),
                        (1,),
                        mode=lax.GatherScatterMode.PROMISE_IN_BOUNDS)
                    e = j * _LANES + e16
                    for g in range(D // _LANES):
                        sl = pl.ds(g * _LANES, _LANES)
                        gx[b][e, sl] = gx[b][e, sl] * vb
                return carry

            lax.fori_loop(0, CH // _LANES, jbody, 0)

        def slot(c, b):
            wait_gather(b)
            bn = (b + 2) % BUFS

            @pl.when(c >= 1)
            def _():
                wait_scatter(bn)

            @pl.when(c + 2 < NCHW)
            def _():
                unpack(c + 2, bn)
                issue_gather(c + 2, bn)

            scale(c, b)
            issue_scatter(b)

        unpack(0, 0)
        issue_gather(0, 0)
        unpack(1, 1)
        issue_gather(1, 1)

        def gbody(g, carry):
            c0 = g * BUFS
            for b in range(BUFS):
                slot(c0 + b, b)
            return carry

        lax.fori_loop(0, GROUPS, gbody, 0)
        for r in range(REM):
            slot(GROUPS * BUFS + r, r)

        wait_scatter((NCHW - 1) % BUFS)
        plsc.subcore_barrier()

        pltpu.sync_copy(acc.at[pl.ds(rbase, RPS)],
                        out_hbm.at[cid, pl.ds(rbase, RPS)])

        @pl.when(sid == NS - 1)
        def _():
            pltpu.sync_copy(acc.at[pl.ds(NS * RPS, TAIL)],
                            out_hbm.at[cid, pl.ds(NS * RPS, TAIL)])

    return spmm_kernel


def _dense(h, p, Wa, Wb, b, relu, res):
    N, D = h.shape
    BN = 1000

    def body(h_ref, p_ref, wa_ref, wb_ref, b_ref, o_ref):
        z = p_ref[0] + p_ref[1]
        acc = jnp.dot(h_ref[...], wa_ref[...],
                      preferred_element_type=jnp.float32)
        acc = acc + jnp.dot(z, wb_ref[...],
                            preferred_element_type=jnp.float32)
        acc = acc + b_ref[...]
        if res:
            acc = acc + h_ref[...]
        if relu:
            acc = jnp.maximum(acc, 0.0)
        o_ref[...] = acc

    return pl.pallas_call(
        body,
        grid=(N // BN,),
        in_specs=[
            pl.BlockSpec((BN, D), lambda i: (i, 0)),
            pl.BlockSpec((2, BN, D), lambda i: (0, i, 0)),
            pl.BlockSpec((D, D), lambda i: (0, 0)),
            pl.BlockSpec((D, D), lambda i: (0, 0)),
            pl.BlockSpec((1, D), lambda i: (0, 0)),
        ],
        out_specs=pl.BlockSpec((BN, D), lambda i: (i, 0)),
        out_shape=jax.ShapeDtypeStruct((N, D), jnp.float32),
    )(h, p, Wa, Wb, b.reshape(1, D))


def kernel(x, laplacian_indices, laplacian_values, W0, b0, W1, b1, W2, b2):
    N, D = x.shape
    E = laplacian_values.shape[0]
    assert N <= (1 << _IDXB)
    rows = laplacian_indices[0]
    cols = laplacian_indices[1]

    CH, NW = 64, 32
    NCHW = -(-E // (NW * CH))
    NCHW = -(-NCHW // 8) * 8
    EP = NW * NCHW * CH
    pad = EP - E
    if pad:
        pidx = (jnp.arange(pad, dtype=jnp.int32) * 37) % N
        rows_p = jnp.concatenate([rows, pidx])
        cols_p = jnp.concatenate([cols, pidx])
        vals_p = jnp.concatenate(
            [laplacian_values, jnp.zeros((pad,), jnp.float32)])
    else:
        rows_p, cols_p, vals_p = rows, cols, laplacian_values
    packed = ((rows_p << _IDXB) | cols_p).reshape(EP // (2 * CH), 2 * CH)
    vals2 = vals_p.reshape(EP // (2 * CH), 2 * CH)

    Wa0, Wb0 = W0[:, 0::2].T, W0[:, 1::2].T
    Wa1, Wb1 = W1[:, 0::2].T, W1[:, 1::2].T
    Wa2, Wb2 = W2[:, 0::2].T, W2[:, 1::2].T

    spmm = _sc_spmm(N, D, EP)

    p = spmm(packed, vals2, x)
    h = _dense(x, p, Wa0, Wb0, b0, relu=True, res=False)
    p = spmm(packed, vals2, h)
    h = _dense(h, p, Wa1, Wb1, b1, relu=True, res=True)
    p = spmm(packed, vals2, h)
    out = _dense(h, p, Wa2, Wb2, b2, relu=False, res=False)
    return out
